# baseline (speedup 1.0000x reference)
import functools

import jax
import jax.numpy as jnp
from jax.experimental import pallas as pl
from jax.experimental.pallas import tpu as pltpu


def _block_kernel(chars_ref, lens_ref, emb_ref, wihT_ref, whhT_f_ref,
                  whhT_b_ref, bih_ref, bhh_f_ref, bhh_b_ref, wpT_ref,
                  bp_ref, ctx_ref, out_ref):
    T, BW, _ = chars_ref.shape
    H = whhT_f_ref.shape[0]
    V = emb_ref.shape[0]

    chars = chars_ref[...]
    lens = lens_ref[...]

    iota_c = jax.lax.broadcasted_iota(jnp.int32, (T, BW, V), 2)
    oh = (chars == iota_c).astype(jnp.float32)
    oh = oh.reshape(T * BW, V).astype(jnp.bfloat16)
    table = jnp.dot(emb_ref[...], wihT_ref[...],
                    preferred_element_type=jnp.float32).astype(jnp.bfloat16)
    xp = jnp.dot(oh, table, preferred_element_type=jnp.float32) \
        + bih_ref[...]
    xp_f = xp[:, :3 * H]
    xp_b = xp[:, 3 * H:]

    def cell(xt, h, whhT, bhh):
        hp = jnp.dot(h.astype(jnp.bfloat16), whhT,
                     preferred_element_type=jnp.float32) + bhh
        r = jax.nn.sigmoid(xt[:, :H] + hp[:, :H])
        z = jax.nn.sigmoid(xt[:, H:2 * H] + hp[:, H:2 * H])
        n = jnp.tanh(xt[:, 2 * H:] + r * hp[:, 2 * H:])
        return (1.0 - z) * n + z * h

    whhT_f = whhT_f_ref[...].astype(jnp.bfloat16)
    whhT_b = whhT_b_ref[...].astype(jnp.bfloat16)
    bhh_f = bhh_f_ref[...]
    bhh_b = bhh_b_ref[...]

    h = jnp.zeros((BW, H), jnp.float32)
    outs_f = []
    for t in range(T):
        h = cell(xp_f[t * BW:(t + 1) * BW, :], h, whhT_f, bhh_f)
        outs_f.append(h)

    h = jnp.zeros((BW, H), jnp.float32)
    outs_b = [None] * T
    for t in range(T - 1, -1, -1):
        hn = cell(xp_b[t * BW:(t + 1) * BW, :], h, whhT_b, bhh_b)
        h = jnp.where(t < lens, hn, h)
        outs_b[t] = h

    rows = []
    for t in range(T):
        oc = jnp.concatenate([outs_f[t], outs_b[t]], axis=1)
        rows.append(jnp.where(t < lens, oc, 0.0))
    ocat = jnp.concatenate(rows, axis=0)

    proj = jnp.tanh(
        jnp.dot(ocat.astype(jnp.bfloat16), wpT_ref[...].astype(jnp.bfloat16),
                preferred_element_type=jnp.float32)
        + bp_ref[...])
    s = jnp.sum(proj * ctx_ref[...], axis=1, keepdims=True)
    s3 = s.reshape(T, BW, 1)
    m = jnp.max(s3, axis=0, keepdims=True)
    e = jnp.exp(s3 - m)
    att = e / jnp.sum(e, axis=0, keepdims=True)
    o3 = ocat.reshape(T, BW, 2 * H) * att
    out_ref[...] = jnp.sum(o3, axis=0)


@functools.partial(jax.jit, static_argnames=("interpret",))
def _char_to_word(padded_char_tensor, sequence_lens, emb, Wih_f, Whh_f,
                  bih_f, bhh_f, Wih_b, Whh_b, bih_b, bhh_b, Wp, bp, ctx,
                  interpret=False):
    NW, T = padded_char_tensor.shape
    V, EMB = emb.shape
    H = Whh_f.shape[1]
    C = Wp.shape[0]
    BW = 256 if NW % 256 == 0 else NW
    n_blocks = NW // BW

    chars3 = padded_char_tensor.astype(jnp.int32).T[:, :, None]
    lens2 = sequence_lens.astype(jnp.int32)[:, None]
    wihT = jnp.concatenate([Wih_f.T, Wih_b.T], axis=1)
    bih = jnp.concatenate([bih_f, bih_b])[None, :]
    out = pl.pallas_call(
        _block_kernel,
        out_shape=jax.ShapeDtypeStruct((NW, 2 * H), jnp.float32),
        grid=(n_blocks,),
        in_specs=[
            pl.BlockSpec((T, BW, 1), lambda i: (0, i, 0)),
            pl.BlockSpec((BW, 1), lambda i: (i, 0)),
            pl.BlockSpec((V, EMB), lambda i: (0, 0)),
            pl.BlockSpec((EMB, 6 * H), lambda i: (0, 0)),
            pl.BlockSpec((H, 3 * H), lambda i: (0, 0)),
            pl.BlockSpec((H, 3 * H), lambda i: (0, 0)),
            pl.BlockSpec((1, 6 * H), lambda i: (0, 0)),
            pl.BlockSpec((1, 3 * H), lambda i: (0, 0)),
            pl.BlockSpec((1, 3 * H), lambda i: (0, 0)),
            pl.BlockSpec((2 * H, C), lambda i: (0, 0)),
            pl.BlockSpec((1, C), lambda i: (0, 0)),
            pl.BlockSpec((1, C), lambda i: (0, 0)),
        ],
        out_specs=pl.BlockSpec((BW, 2 * H), lambda i: (i, 0)),
        compiler_params=pltpu.CompilerParams(
            dimension_semantics=("parallel",),
            vmem_limit_bytes=50 * 1024 * 1024,
        ),
        name="char_to_word",
        interpret=interpret,
    )(
        chars3, lens2, emb, wihT, Whh_f.T, Whh_b.T, bih,
        bhh_f[None, :], bhh_b[None, :], Wp.T, bp[None, :], ctx.T,
    )
    return out


def kernel(padded_char_tensor, sequence_lens, emb, Wih_f, Whh_f, bih_f,
           bhh_f, Wih_b, Whh_b, bih_b, bhh_b, Wp, bp, ctx):
    return _char_to_word(padded_char_tensor, sequence_lens, emb, Wih_f,
                         Whh_f, bih_f, bhh_f, Wih_b, Whh_b, bih_b, bhh_b,
                         Wp, bp, ctx)


# speedup vs baseline: 1.2377x; 1.2377x over previous
import functools

import jax
import jax.numpy as jnp
from jax.experimental import pallas as pl
from jax.experimental.pallas import tpu as pltpu


def _block_kernel(chars_ref, lens_ref, emb_ref, wihT_ref, whhT_f_ref,
                  whhT_b_ref, bih_ref, bhh_f_ref, bhh_b_ref, wpT_ref,
                  bp_ref, ctx_ref, out_ref):
    T, BW, _ = chars_ref.shape
    H = whhT_f_ref.shape[0]
    V = emb_ref.shape[0]

    chars = chars_ref[...]
    lens = lens_ref[...]

    iota_c = jax.lax.broadcasted_iota(jnp.int32, (T, BW, V), 2)
    oh = (chars == iota_c).astype(jnp.float32)
    oh = oh.reshape(T * BW, V).astype(jnp.bfloat16)
    table = jnp.dot(emb_ref[...], wihT_ref[...],
                    preferred_element_type=jnp.float32).astype(jnp.bfloat16)
    xp = jnp.dot(oh, table, preferred_element_type=jnp.float32) \
        + bih_ref[...]
    xp_f = xp[:, :3 * H]
    xp_b = xp[:, 3 * H:]

    def cell(xt, h, whhT, bhh):
        hp = jnp.dot(h.astype(jnp.bfloat16), whhT,
                     preferred_element_type=jnp.float32) + bhh
        r = 0.5 * jnp.tanh(0.5 * (xt[:, :H] + hp[:, :H])) + 0.5
        z = 0.5 * jnp.tanh(0.5 * (xt[:, H:2 * H] + hp[:, H:2 * H])) + 0.5
        n = jnp.tanh(xt[:, 2 * H:] + r * hp[:, 2 * H:])
        return n + z * (h - n)

    whhT_f = whhT_f_ref[...].astype(jnp.bfloat16)
    whhT_b = whhT_b_ref[...].astype(jnp.bfloat16)
    bhh_f = bhh_f_ref[...]
    bhh_b = bhh_b_ref[...]

    h = jnp.zeros((BW, H), jnp.float32)
    outs_f = []
    for t in range(T):
        h = cell(xp_f[t * BW:(t + 1) * BW, :], h, whhT_f, bhh_f)
        outs_f.append(h)

    h = jnp.zeros((BW, H), jnp.float32)
    outs_b = [None] * T
    for t in range(T - 1, -1, -1):
        hn = cell(xp_b[t * BW:(t + 1) * BW, :], h, whhT_b, bhh_b)
        h = jnp.where(t < lens, hn, h)
        outs_b[t] = h

    rows = []
    for t in range(T):
        oc = jnp.concatenate([outs_f[t], outs_b[t]], axis=1)
        rows.append(jnp.where(t < lens, oc, 0.0))
    ocat = jnp.concatenate(rows, axis=0)

    proj = jnp.tanh(
        jnp.dot(ocat.astype(jnp.bfloat16), wpT_ref[...].astype(jnp.bfloat16),
                preferred_element_type=jnp.float32)
        + bp_ref[...])
    s = jnp.sum(proj * ctx_ref[...], axis=1, keepdims=True)
    s3 = s.reshape(T, BW, 1)
    m = jnp.max(s3, axis=0, keepdims=True)
    e = jnp.exp(s3 - m)
    att = e / jnp.sum(e, axis=0, keepdims=True)
    o3 = ocat.reshape(T, BW, 2 * H) * att
    out_ref[...] = jnp.sum(o3, axis=0)


@functools.partial(jax.jit, static_argnames=("interpret",))
def _char_to_word(padded_char_tensor, sequence_lens, emb, Wih_f, Whh_f,
                  bih_f, bhh_f, Wih_b, Whh_b, bih_b, bhh_b, Wp, bp, ctx,
                  interpret=False):
    NW, T = padded_char_tensor.shape
    V, EMB = emb.shape
    H = Whh_f.shape[1]
    C = Wp.shape[0]
    BW = 512 if NW % 512 == 0 else NW
    n_blocks = NW // BW

    chars3 = padded_char_tensor.astype(jnp.int32).T[:, :, None]
    lens2 = sequence_lens.astype(jnp.int32)[:, None]
    wihT = jnp.concatenate([Wih_f.T, Wih_b.T], axis=1)
    bih = jnp.concatenate([bih_f, bih_b])[None, :]
    out = pl.pallas_call(
        _block_kernel,
        out_shape=jax.ShapeDtypeStruct((NW, 2 * H), jnp.float32),
        grid=(n_blocks,),
        in_specs=[
            pl.BlockSpec((T, BW, 1), lambda i: (0, i, 0)),
            pl.BlockSpec((BW, 1), lambda i: (i, 0)),
            pl.BlockSpec((V, EMB), lambda i: (0, 0)),
            pl.BlockSpec((EMB, 6 * H), lambda i: (0, 0)),
            pl.BlockSpec((H, 3 * H), lambda i: (0, 0)),
            pl.BlockSpec((H, 3 * H), lambda i: (0, 0)),
            pl.BlockSpec((1, 6 * H), lambda i: (0, 0)),
            pl.BlockSpec((1, 3 * H), lambda i: (0, 0)),
            pl.BlockSpec((1, 3 * H), lambda i: (0, 0)),
            pl.BlockSpec((2 * H, C), lambda i: (0, 0)),
            pl.BlockSpec((1, C), lambda i: (0, 0)),
            pl.BlockSpec((1, C), lambda i: (0, 0)),
        ],
        out_specs=pl.BlockSpec((BW, 2 * H), lambda i: (i, 0)),
        compiler_params=pltpu.CompilerParams(
            dimension_semantics=("parallel",),
            vmem_limit_bytes=50 * 1024 * 1024,
        ),
        name="char_to_word",
        interpret=interpret,
    )(
        chars3, lens2, emb, wihT, Whh_f.T, Whh_b.T, bih,
        bhh_f[None, :], bhh_b[None, :], Wp.T, bp[None, :], ctx.T,
    )
    return out


def kernel(padded_char_tensor, sequence_lens, emb, Wih_f, Whh_f, bih_f,
           bhh_f, Wih_b, Whh_b, bih_b, bhh_b, Wp, bp, ctx):
    return _char_to_word(padded_char_tensor, sequence_lens, emb, Wih_f,
                         Whh_f, bih_f, bhh_f, Wih_b, Whh_b, bih_b, bhh_b,
                         Wp, bp, ctx)


# speedup vs baseline: 1.3708x; 1.1076x over previous
import functools

import jax
import jax.numpy as jnp
from jax.experimental import pallas as pl
from jax.experimental.pallas import tpu as pltpu


def _block_kernel(chars_ref, lens_ref, emb_ref, wihT_ref, whhT_f_ref,
                  whhT_b_ref, bih_ref, bhh_f_ref, bhh_b_ref, wpT_ref,
                  bp_ref, ctx_ref, out_ref):
    T, BW, _ = chars_ref.shape
    H = whhT_f_ref.shape[0]
    V = emb_ref.shape[0]

    chars = chars_ref[...]
    lens = lens_ref[...]

    iota_c = jax.lax.broadcasted_iota(jnp.int32, (T, BW, V), 2)
    oh = (chars == iota_c).astype(jnp.float32)
    oh = oh.reshape(T * BW, V).astype(jnp.bfloat16)
    table = (jnp.dot(emb_ref[...], wihT_ref[...],
                     preferred_element_type=jnp.float32)
             + bih_ref[...]).astype(jnp.bfloat16)
    xp = jnp.dot(oh, table, preferred_element_type=jnp.float32)
    xp_f = xp[:, :3 * H]
    xp_b = xp[:, 3 * H:]

    aug_iota = jax.lax.broadcasted_iota(jnp.int32, (BW, H), 1)
    aug_ones = (aug_iota == 0).astype(jnp.float32).astype(jnp.bfloat16)

    def aug_w(whhT_ref, bhh_ref):
        return jnp.concatenate(
            [whhT_ref[...], bhh_ref[...],
             jnp.zeros((H - 1, 3 * H), jnp.float32)],
            axis=0).astype(jnp.bfloat16)

    def cell(xt, h, whhA):
        ha = jnp.concatenate([h.astype(jnp.bfloat16), aug_ones], axis=1)
        hp = jnp.dot(ha, whhA, preferred_element_type=jnp.float32)
        r = 0.5 * jnp.tanh(0.5 * (xt[:, :H] + hp[:, :H])) + 0.5
        z = 0.5 * jnp.tanh(0.5 * (xt[:, H:2 * H] + hp[:, H:2 * H])) + 0.5
        n = jnp.tanh(xt[:, 2 * H:] + r * hp[:, 2 * H:])
        return n + z * (h - n)

    whhA_f = aug_w(whhT_f_ref, bhh_f_ref)
    whhA_b = aug_w(whhT_b_ref, bhh_b_ref)

    h = jnp.zeros((BW, H), jnp.float32)
    outs_f = []
    for t in range(T):
        h = cell(xp_f[t * BW:(t + 1) * BW, :], h, whhA_f)
        outs_f.append(h)

    h = jnp.zeros((BW, H), jnp.float32)
    outs_b = [None] * T
    for t in range(T - 1, -1, -1):
        hn = cell(xp_b[t * BW:(t + 1) * BW, :], h, whhA_b)
        h = jnp.where(t < lens, hn, h)
        outs_b[t] = h

    rows = []
    for t in range(T):
        oc = jnp.concatenate([outs_f[t], outs_b[t]], axis=1)
        rows.append(jnp.where(t < lens, oc, 0.0))
    ocat = jnp.concatenate(rows, axis=0)

    proj = jnp.tanh(
        jnp.dot(ocat.astype(jnp.bfloat16), wpT_ref[...].astype(jnp.bfloat16),
                preferred_element_type=jnp.float32)
        + bp_ref[...])
    s = jnp.sum(proj * ctx_ref[...], axis=1, keepdims=True)
    s3 = s.reshape(T, BW, 1)
    m = jnp.max(s3, axis=0, keepdims=True)
    e = jnp.exp(s3 - m)
    att = e / jnp.sum(e, axis=0, keepdims=True)
    o3 = ocat.reshape(T, BW, 2 * H) * att
    out_ref[...] = jnp.sum(o3, axis=0)


@functools.partial(jax.jit, static_argnames=("interpret",))
def _char_to_word(padded_char_tensor, sequence_lens, emb, Wih_f, Whh_f,
                  bih_f, bhh_f, Wih_b, Whh_b, bih_b, bhh_b, Wp, bp, ctx,
                  interpret=False):
    NW, T = padded_char_tensor.shape
    V, EMB = emb.shape
    H = Whh_f.shape[1]
    C = Wp.shape[0]
    BW = 512 if NW % 512 == 0 else NW
    n_blocks = NW // BW

    chars3 = padded_char_tensor.astype(jnp.int32).T[:, :, None]
    lens2 = sequence_lens.astype(jnp.int32)[:, None]
    wihT = jnp.concatenate([Wih_f.T, Wih_b.T], axis=1)
    bih = jnp.concatenate([bih_f, bih_b])[None, :]
    out = pl.pallas_call(
        _block_kernel,
        out_shape=jax.ShapeDtypeStruct((NW, 2 * H), jnp.float32),
        grid=(n_blocks,),
        in_specs=[
            pl.BlockSpec((T, BW, 1), lambda i: (0, i, 0)),
            pl.BlockSpec((BW, 1), lambda i: (i, 0)),
            pl.BlockSpec((V, EMB), lambda i: (0, 0)),
            pl.BlockSpec((EMB, 6 * H), lambda i: (0, 0)),
            pl.BlockSpec((H, 3 * H), lambda i: (0, 0)),
            pl.BlockSpec((H, 3 * H), lambda i: (0, 0)),
            pl.BlockSpec((1, 6 * H), lambda i: (0, 0)),
            pl.BlockSpec((1, 3 * H), lambda i: (0, 0)),
            pl.BlockSpec((1, 3 * H), lambda i: (0, 0)),
            pl.BlockSpec((2 * H, C), lambda i: (0, 0)),
            pl.BlockSpec((1, C), lambda i: (0, 0)),
            pl.BlockSpec((1, C), lambda i: (0, 0)),
        ],
        out_specs=pl.BlockSpec((BW, 2 * H), lambda i: (i, 0)),
        compiler_params=pltpu.CompilerParams(
            dimension_semantics=("parallel",),
            vmem_limit_bytes=50 * 1024 * 1024,
        ),
        name="char_to_word",
        interpret=interpret,
    )(
        chars3, lens2, emb, wihT, Whh_f.T, Whh_b.T, bih,
        bhh_f[None, :], bhh_b[None, :], Wp.T, bp[None, :], ctx.T,
    )
    return out


def kernel(padded_char_tensor, sequence_lens, emb, Wih_f, Whh_f, bih_f,
           bhh_f, Wih_b, Whh_b, bih_b, bhh_b, Wp, bp, ctx):
    return _char_to_word(padded_char_tensor, sequence_lens, emb, Wih_f,
                         Whh_f, bih_f, bhh_f, Wih_b, Whh_b, bih_b, bhh_b,
                         Wp, bp, ctx)


# speedup vs baseline: 1.4082x; 1.0273x over previous
import functools

import jax
import jax.numpy as jnp
from jax.experimental import pallas as pl
from jax.experimental.pallas import tpu as pltpu


def _block_kernel(chars_ref, lens_ref, emb_ref, wihT_ref, whhT_f_ref,
                  whhT_b_ref, bih_ref, bhh_f_ref, bhh_b_ref, wpT_ref,
                  bp_ref, ctx_ref, out_ref):
    T, BW, _ = chars_ref.shape
    H = whhT_f_ref.shape[0]
    V = emb_ref.shape[0]

    chars = chars_ref[...]
    lens = lens_ref[...]

    iota_c = jax.lax.broadcasted_iota(jnp.int32, (T, BW, V), 2)
    oh = (chars == iota_c).astype(jnp.float32)
    oh = oh.reshape(T * BW, V).astype(jnp.bfloat16)
    table = (jnp.dot(emb_ref[...], wihT_ref[...],
                     preferred_element_type=jnp.float32)
             + bih_ref[...]).astype(jnp.bfloat16)
    xp = jnp.dot(oh, table, preferred_element_type=jnp.float32)
    xp_f = xp[:, :3 * H]
    xp_b = xp[:, 3 * H:]

    aug_iota = jax.lax.broadcasted_iota(jnp.int32, (BW, H), 1)
    aug_ones = (aug_iota == 0).astype(jnp.float32).astype(jnp.bfloat16)

    def aug_w(whhT_ref, bhh_ref):
        return jnp.concatenate(
            [whhT_ref[...], bhh_ref[...],
             jnp.zeros((H - 1, 3 * H), jnp.float32)],
            axis=0).astype(jnp.bfloat16)

    def cell(xt, h, hb, whhA):
        ha = jnp.concatenate([hb, aug_ones], axis=1)
        hp = jnp.dot(ha, whhA, preferred_element_type=jnp.float32)
        r = 0.5 * jnp.tanh(0.5 * (xt[:, :H] + hp[:, :H])) + 0.5
        z = 0.5 * jnp.tanh(0.5 * (xt[:, H:2 * H] + hp[:, H:2 * H])) + 0.5
        n = jnp.tanh(xt[:, 2 * H:] + r * hp[:, 2 * H:])
        return n + z * (h - n)

    whhA_f = aug_w(whhT_f_ref, bhh_f_ref)
    whhA_b = aug_w(whhT_b_ref, bhh_b_ref)
    zero_b = jnp.zeros((BW, H), jnp.bfloat16)

    h = jnp.zeros((BW, H), jnp.float32)
    hb = zero_b
    outs_f = []
    for t in range(T):
        h = cell(xp_f[t * BW:(t + 1) * BW, :], h, hb, whhA_f)
        hb = h.astype(jnp.bfloat16)
        outs_f.append(hb)

    h = jnp.zeros((BW, H), jnp.float32)
    hb = zero_b
    outs_b = [None] * T
    for t in range(T - 1, -1, -1):
        hn = cell(xp_b[t * BW:(t + 1) * BW, :], h, hb, whhA_b)
        h = jnp.where(t < lens, hn, h)
        hb = h.astype(jnp.bfloat16)
        outs_b[t] = hb

    rows = []
    for t in range(T):
        oc = jnp.concatenate([outs_f[t], outs_b[t]], axis=1)
        rows.append(jnp.where(t < lens, oc, jnp.bfloat16(0)))
    ocat = jnp.concatenate(rows, axis=0)

    proj = jnp.tanh(
        jnp.dot(ocat, wpT_ref[...].astype(jnp.bfloat16),
                preferred_element_type=jnp.float32)
        + bp_ref[...])
    s = jnp.sum(proj * ctx_ref[...], axis=1, keepdims=True)
    s3 = s.reshape(T, BW, 1)
    m = jnp.max(s3, axis=0, keepdims=True)
    e = jnp.exp(s3 - m)
    att = e / jnp.sum(e, axis=0, keepdims=True)
    o3 = ocat.reshape(T, BW, 2 * H).astype(jnp.float32) * att
    out_ref[...] = jnp.sum(o3, axis=0)


@functools.partial(jax.jit, static_argnames=("interpret",))
def _char_to_word(padded_char_tensor, sequence_lens, emb, Wih_f, Whh_f,
                  bih_f, bhh_f, Wih_b, Whh_b, bih_b, bhh_b, Wp, bp, ctx,
                  interpret=False):
    NW, T = padded_char_tensor.shape
    V, EMB = emb.shape
    H = Whh_f.shape[1]
    C = Wp.shape[0]
    BW = 512 if NW % 512 == 0 else NW
    n_blocks = NW // BW

    chars3 = padded_char_tensor.astype(jnp.int32).T[:, :, None]
    lens2 = sequence_lens.astype(jnp.int32)[:, None]
    wihT = jnp.concatenate([Wih_f.T, Wih_b.T], axis=1)
    bih = jnp.concatenate([bih_f, bih_b])[None, :]
    out = pl.pallas_call(
        _block_kernel,
        out_shape=jax.ShapeDtypeStruct((NW, 2 * H), jnp.float32),
        grid=(n_blocks,),
        in_specs=[
            pl.BlockSpec((T, BW, 1), lambda i: (0, i, 0)),
            pl.BlockSpec((BW, 1), lambda i: (i, 0)),
            pl.BlockSpec((V, EMB), lambda i: (0, 0)),
            pl.BlockSpec((EMB, 6 * H), lambda i: (0, 0)),
            pl.BlockSpec((H, 3 * H), lambda i: (0, 0)),
            pl.BlockSpec((H, 3 * H), lambda i: (0, 0)),
            pl.BlockSpec((1, 6 * H), lambda i: (0, 0)),
            pl.BlockSpec((1, 3 * H), lambda i: (0, 0)),
            pl.BlockSpec((1, 3 * H), lambda i: (0, 0)),
            pl.BlockSpec((2 * H, C), lambda i: (0, 0)),
            pl.BlockSpec((1, C), lambda i: (0, 0)),
            pl.BlockSpec((1, C), lambda i: (0, 0)),
        ],
        out_specs=pl.BlockSpec((BW, 2 * H), lambda i: (i, 0)),
        compiler_params=pltpu.CompilerParams(
            dimension_semantics=("parallel",),
            vmem_limit_bytes=50 * 1024 * 1024,
        ),
        name="char_to_word",
        interpret=interpret,
    )(
        chars3, lens2, emb, wihT, Whh_f.T, Whh_b.T, bih,
        bhh_f[None, :], bhh_b[None, :], Wp.T, bp[None, :], ctx.T,
    )
    return out


def kernel(padded_char_tensor, sequence_lens, emb, Wih_f, Whh_f, bih_f,
           bhh_f, Wih_b, Whh_b, bih_b, bhh_b, Wp, bp, ctx):
    return _char_to_word(padded_char_tensor, sequence_lens, emb, Wih_f,
                         Whh_f, bih_f, bhh_f, Wih_b, Whh_b, bih_b, bhh_b,
                         Wp, bp, ctx)


# speedup vs baseline: 1.4251x; 1.0120x over previous
import functools

import jax
import jax.numpy as jnp
from jax.experimental import pallas as pl
from jax.experimental.pallas import tpu as pltpu


def _block_kernel(chars_ref, lens_ref, emb_ref, wihT_ref, whhT_f_ref,
                  whhT_b_ref, bih_ref, bhh_f_ref, bhh_b_ref, wpT_ref,
                  bp_ref, ctx_ref, out_ref):
    T, BW, _ = chars_ref.shape
    H = whhT_f_ref.shape[0]
    V = emb_ref.shape[0]

    chars = chars_ref[...]
    lens = lens_ref[...]

    iota_c = jax.lax.broadcasted_iota(jnp.int32, (T, BW, V), 2)
    oh = (chars == iota_c).astype(jnp.float32)
    oh = oh.reshape(T * BW, V).astype(jnp.bfloat16)
    table = (jnp.dot(emb_ref[...], wihT_ref[...],
                     preferred_element_type=jnp.float32)
             + bih_ref[...]).astype(jnp.bfloat16)
    xp = jnp.dot(oh, table, preferred_element_type=jnp.float32)
    xp_f = xp[:, :3 * H]
    xp_b = xp[:, 3 * H:]

    aug_iota = jax.lax.broadcasted_iota(jnp.int32, (BW, H), 1)
    aug_ones = (aug_iota == 0).astype(jnp.float32).astype(jnp.bfloat16)

    def aug_w(whhT_ref, bhh_ref):
        return jnp.concatenate(
            [whhT_ref[...], bhh_ref[...],
             jnp.zeros((H - 1, 3 * H), jnp.float32)],
            axis=0).astype(jnp.bfloat16)

    def cell(xt, h, hb, whhA):
        ha = jnp.concatenate([hb, aug_ones], axis=1)
        hp = jnp.dot(ha, whhA, preferred_element_type=jnp.float32)
        r = 0.5 * jnp.tanh(0.5 * (xt[:, :H] + hp[:, :H])) + 0.5
        z = 0.5 * jnp.tanh(0.5 * (xt[:, H:2 * H] + hp[:, H:2 * H])) + 0.5
        n = jnp.tanh(xt[:, 2 * H:] + r * hp[:, 2 * H:])
        return n + z * (h - n)

    whhA_f = aug_w(whhT_f_ref, bhh_f_ref)
    whhA_b = aug_w(whhT_b_ref, bhh_b_ref)
    zero_b = jnp.zeros((BW, H), jnp.bfloat16)

    h = jnp.zeros((BW, H), jnp.float32)
    hb = zero_b
    outs_f = []
    for t in range(T):
        h = cell(xp_f[t * BW:(t + 1) * BW, :], h, hb, whhA_f)
        hb = h.astype(jnp.bfloat16)
        outs_f.append(jnp.where(t < lens, h, 0.0).astype(jnp.bfloat16))

    h = jnp.zeros((BW, H), jnp.float32)
    hb = zero_b
    outs_b = [None] * T
    for t in range(T - 1, -1, -1):
        hn = cell(xp_b[t * BW:(t + 1) * BW, :], h, hb, whhA_b)
        h = jnp.where(t < lens, hn, h)
        hb = h.astype(jnp.bfloat16)
        outs_b[t] = hb

    rows = []
    for t in range(T):
        rows.append(jnp.concatenate([outs_f[t], outs_b[t]], axis=1))
    ocat = jnp.concatenate(rows, axis=0)

    proj = jnp.tanh(
        jnp.dot(ocat, wpT_ref[...].astype(jnp.bfloat16),
                preferred_element_type=jnp.float32)
        + bp_ref[...])
    s = jnp.sum(proj * ctx_ref[...], axis=1, keepdims=True)
    s3 = s.reshape(T, BW, 1)
    e = jnp.exp(s3)
    att = e / jnp.sum(e, axis=0, keepdims=True)
    o3 = ocat.reshape(T, BW, 2 * H).astype(jnp.float32) * att
    out_ref[...] = jnp.sum(o3, axis=0)


@functools.partial(jax.jit, static_argnames=("interpret",))
def _char_to_word(padded_char_tensor, sequence_lens, emb, Wih_f, Whh_f,
                  bih_f, bhh_f, Wih_b, Whh_b, bih_b, bhh_b, Wp, bp, ctx,
                  interpret=False):
    NW, T = padded_char_tensor.shape
    V, EMB = emb.shape
    H = Whh_f.shape[1]
    C = Wp.shape[0]
    BW = 512 if NW % 512 == 0 else NW
    n_blocks = NW // BW

    chars3 = padded_char_tensor.astype(jnp.int32).T[:, :, None]
    lens2 = sequence_lens.astype(jnp.int32)[:, None]
    wihT = jnp.concatenate([Wih_f.T, Wih_b.T], axis=1)
    bih = jnp.concatenate([bih_f, bih_b])[None, :]
    out = pl.pallas_call(
        _block_kernel,
        out_shape=jax.ShapeDtypeStruct((NW, 2 * H), jnp.float32),
        grid=(n_blocks,),
        in_specs=[
            pl.BlockSpec((T, BW, 1), lambda i: (0, i, 0)),
            pl.BlockSpec((BW, 1), lambda i: (i, 0)),
            pl.BlockSpec((V, EMB), lambda i: (0, 0)),
            pl.BlockSpec((EMB, 6 * H), lambda i: (0, 0)),
            pl.BlockSpec((H, 3 * H), lambda i: (0, 0)),
            pl.BlockSpec((H, 3 * H), lambda i: (0, 0)),
            pl.BlockSpec((1, 6 * H), lambda i: (0, 0)),
            pl.BlockSpec((1, 3 * H), lambda i: (0, 0)),
            pl.BlockSpec((1, 3 * H), lambda i: (0, 0)),
            pl.BlockSpec((2 * H, C), lambda i: (0, 0)),
            pl.BlockSpec((1, C), lambda i: (0, 0)),
            pl.BlockSpec((1, C), lambda i: (0, 0)),
        ],
        out_specs=pl.BlockSpec((BW, 2 * H), lambda i: (i, 0)),
        compiler_params=pltpu.CompilerParams(
            dimension_semantics=("parallel",),
            vmem_limit_bytes=50 * 1024 * 1024,
        ),
        name="char_to_word",
        interpret=interpret,
    )(
        chars3, lens2, emb, wihT, Whh_f.T, Whh_b.T, bih,
        bhh_f[None, :], bhh_b[None, :], Wp.T, bp[None, :], ctx.T,
    )
    return out


def kernel(padded_char_tensor, sequence_lens, emb, Wih_f, Whh_f, bih_f,
           bhh_f, Wih_b, Whh_b, bih_b, bhh_b, Wp, bp, ctx):
    return _char_to_word(padded_char_tensor, sequence_lens, emb, Wih_f,
                         Whh_f, bih_f, bhh_f, Wih_b, Whh_b, bih_b, bhh_b,
                         Wp, bp, ctx)


# speedup vs baseline: 1.7045x; 1.1961x over previous
import functools

import jax
import jax.numpy as jnp
from jax.experimental import pallas as pl
from jax.experimental.pallas import tpu as pltpu


def _block_kernel(chars_ref, lens_ref, emb_ref, wihT_ref, whhT_f_ref,
                  whhT_b_ref, bih_ref, bhh_f_ref, bhh_b_ref, wpT_ref,
                  bp_ref, ctx_ref, out_ref):
    BW, T = chars_ref.shape
    H = whhT_f_ref.shape[0]
    V = emb_ref.shape[0]

    ch = chars_ref[...]
    lens = lens_ref[...]

    iota_c = jax.lax.broadcasted_iota(jnp.int32, (BW, V), 1)
    oh = jnp.concatenate(
        [(ch[:, t:t + 1] == iota_c).astype(jnp.float32) for t in range(T)],
        axis=0).astype(jnp.bfloat16)
    table = (jnp.dot(emb_ref[...], wihT_ref[...],
                     preferred_element_type=jnp.float32)
             + bih_ref[...]).astype(jnp.bfloat16)
    xp_f = jnp.dot(oh, table[:, :3 * H],
                   preferred_element_type=jnp.float32)

    aug_iota = jax.lax.broadcasted_iota(jnp.int32, (BW, H), 1)
    aug_ones = (aug_iota == 0).astype(jnp.float32).astype(jnp.bfloat16)

    def aug_w(whhT_ref, bhh_ref):
        return jnp.concatenate(
            [whhT_ref[...], bhh_ref[...],
             jnp.zeros((H - 1, 3 * H), jnp.float32)],
            axis=0).astype(jnp.bfloat16)

    def cell(xt, h, hb, whhA):
        ha = jnp.concatenate([hb, aug_ones], axis=1)
        hp = jnp.dot(ha, whhA, preferred_element_type=jnp.float32)
        r = 0.5 * jnp.tanh(0.5 * (xt[:, :H] + hp[:, :H])) + 0.5
        z = 0.5 * jnp.tanh(0.5 * (xt[:, H:2 * H] + hp[:, H:2 * H])) + 0.5
        n = jnp.tanh(xt[:, 2 * H:] + r * hp[:, 2 * H:])
        return n + z * (h - n)

    whhA_f = aug_w(whhT_f_ref, bhh_f_ref)
    whhA_b = aug_w(whhT_b_ref, bhh_b_ref)
    zero_b = jnp.zeros((BW, H), jnp.bfloat16)

    h = jnp.zeros((BW, H), jnp.float32)
    hb = zero_b
    outs_f = []
    for t in range(T):
        h = cell(xp_f[t * BW:(t + 1) * BW, :], h, hb, whhA_f)
        hb = h.astype(jnp.bfloat16)
        outs_f.append(jnp.where(t < lens, h, 0.0).astype(jnp.bfloat16))

    xp_b = jnp.dot(oh, table[:, 3 * H:],
                   preferred_element_type=jnp.float32)
    h = jnp.zeros((BW, H), jnp.float32)
    hb = zero_b
    outs_b = [None] * T
    for t in range(T - 1, -1, -1):
        hn = cell(xp_b[t * BW:(t + 1) * BW, :], h, hb, whhA_b)
        h = jnp.where(t < lens, hn, h)
        hb = h.astype(jnp.bfloat16)
        outs_b[t] = hb

    wpTb = wpT_ref[...].astype(jnp.bfloat16)
    bp = bp_ref[...]
    ctxr = ctx_ref[...]
    es = []
    for t in range(T):
        oc = jnp.concatenate([outs_f[t], outs_b[t]], axis=1)
        p = jnp.tanh(jnp.dot(oc, wpTb, preferred_element_type=jnp.float32)
                     + bp)
        s_t = jnp.sum(p * ctxr, axis=1, keepdims=True)
        es.append(jnp.exp(s_t))
    den = es[0]
    for t in range(1, T):
        den = den + es[t]
    inv = 1.0 / den
    att0 = es[0] * inv
    acc_f = outs_f[0].astype(jnp.float32) * att0
    acc_b = outs_b[0].astype(jnp.float32) * att0
    for t in range(1, T):
        att_t = es[t] * inv
        acc_f = acc_f + outs_f[t].astype(jnp.float32) * att_t
        acc_b = acc_b + outs_b[t].astype(jnp.float32) * att_t
    out_ref[...] = jnp.concatenate([acc_f, acc_b], axis=1)


@functools.partial(jax.jit, static_argnames=("interpret",))
def _char_to_word(padded_char_tensor, sequence_lens, emb, Wih_f, Whh_f,
                  bih_f, bhh_f, Wih_b, Whh_b, bih_b, bhh_b, Wp, bp, ctx,
                  interpret=False):
    NW, T = padded_char_tensor.shape
    V, EMB = emb.shape
    H = Whh_f.shape[1]
    C = Wp.shape[0]
    BW = 512 if NW % 512 == 0 else NW
    n_blocks = NW // BW

    chars2 = padded_char_tensor.astype(jnp.int32)
    lens2 = sequence_lens.astype(jnp.int32)[:, None]
    wihT = jnp.concatenate([Wih_f.T, Wih_b.T], axis=1)
    bih = jnp.concatenate([bih_f, bih_b])[None, :]
    out = pl.pallas_call(
        _block_kernel,
        out_shape=jax.ShapeDtypeStruct((NW, 2 * H), jnp.float32),
        grid=(n_blocks,),
        in_specs=[
            pl.BlockSpec((BW, T), lambda i: (i, 0)),
            pl.BlockSpec((BW, 1), lambda i: (i, 0)),
            pl.BlockSpec((V, EMB), lambda i: (0, 0)),
            pl.BlockSpec((EMB, 6 * H), lambda i: (0, 0)),
            pl.BlockSpec((H, 3 * H), lambda i: (0, 0)),
            pl.BlockSpec((H, 3 * H), lambda i: (0, 0)),
            pl.BlockSpec((1, 6 * H), lambda i: (0, 0)),
            pl.BlockSpec((1, 3 * H), lambda i: (0, 0)),
            pl.BlockSpec((1, 3 * H), lambda i: (0, 0)),
            pl.BlockSpec((2 * H, C), lambda i: (0, 0)),
            pl.BlockSpec((1, C), lambda i: (0, 0)),
            pl.BlockSpec((1, C), lambda i: (0, 0)),
        ],
        out_specs=pl.BlockSpec((BW, 2 * H), lambda i: (i, 0)),
        compiler_params=pltpu.CompilerParams(
            dimension_semantics=("parallel",),
            vmem_limit_bytes=56 * 1024 * 1024,
        ),
        name="char_to_word",
        interpret=interpret,
    )(
        chars2, lens2, emb, wihT, Whh_f.T, Whh_b.T, bih,
        bhh_f[None, :], bhh_b[None, :], Wp.T, bp[None, :], ctx.T,
    )
    return out


def kernel(padded_char_tensor, sequence_lens, emb, Wih_f, Whh_f, bih_f,
           bhh_f, Wih_b, Whh_b, bih_b, bhh_b, Wp, bp, ctx):
    return _char_to_word(padded_char_tensor, sequence_lens, emb, Wih_f,
                         Whh_f, bih_f, bhh_f, Wih_b, Whh_b, bih_b, bhh_b,
                         Wp, bp, ctx)


# speedup vs baseline: 1.9983x; 1.1724x over previous
import functools

import jax
import jax.numpy as jnp
from jax.experimental import pallas as pl
from jax.experimental.pallas import tpu as pltpu


def _block_kernel(chars_ref, lens_ref, emb_ref, wihT_ref, whhT_f_ref,
                  whhT_b_ref, bih_ref, bhh_f_ref, bhh_b_ref, wpT_ref,
                  bp_ref, ctx_ref, out_ref):
    BW, T = chars_ref.shape
    H = whhT_f_ref.shape[0]
    V = emb_ref.shape[0]

    ch = chars_ref[...]
    lens = lens_ref[...]

    iota_c = jax.lax.broadcasted_iota(jnp.int32, (BW, V), 1)
    ohs = [(ch[:, t:t + 1] == iota_c).astype(jnp.float32).astype(jnp.bfloat16)
           for t in range(T)]
    table = (jnp.dot(emb_ref[...], wihT_ref[...],
                     preferred_element_type=jnp.float32)
             + bih_ref[...]).astype(jnp.bfloat16)
    table_f = table[:, :3 * H]
    table_b = table[:, 3 * H:]

    aug_iota = jax.lax.broadcasted_iota(jnp.int32, (BW, H), 1)
    aug_ones = (aug_iota == 0).astype(jnp.float32).astype(jnp.bfloat16)

    def aug_w(whhT_ref, bhh_ref):
        return jnp.concatenate(
            [whhT_ref[...], bhh_ref[...],
             jnp.zeros((H - 1, 3 * H), jnp.float32)],
            axis=0).astype(jnp.bfloat16)

    def cell(xt, h, hb, whhA):
        ha = jnp.concatenate([hb, aug_ones], axis=1)
        hp = jnp.dot(ha, whhA, preferred_element_type=jnp.float32)
        r = 0.5 * jnp.tanh(0.5 * (xt[:, :H] + hp[:, :H])) + 0.5
        z = 0.5 * jnp.tanh(0.5 * (xt[:, H:2 * H] + hp[:, H:2 * H])) + 0.5
        n = jnp.tanh(xt[:, 2 * H:] + r * hp[:, 2 * H:])
        return n + z * (h - n)

    whhA_f = aug_w(whhT_f_ref, bhh_f_ref)
    whhA_b = aug_w(whhT_b_ref, bhh_b_ref)
    zero_b = jnp.zeros((BW, H), jnp.bfloat16)

    h = jnp.zeros((BW, H), jnp.float32)
    hb = zero_b
    outs_f = []
    for t in range(T):
        xt = jnp.dot(ohs[t], table_f, preferred_element_type=jnp.float32)
        h = cell(xt, h, hb, whhA_f)
        hb = h.astype(jnp.bfloat16)
        outs_f.append(jnp.where(t < lens, h, 0.0).astype(jnp.bfloat16))

    h = jnp.zeros((BW, H), jnp.float32)
    hb = zero_b
    outs_b = [None] * T
    for t in range(T - 1, -1, -1):
        xt = jnp.dot(ohs[t], table_b, preferred_element_type=jnp.float32)
        hn = cell(xt, h, hb, whhA_b)
        h = jnp.where(t < lens, hn, h)
        hb = h.astype(jnp.bfloat16)
        outs_b[t] = hb

    wpTb = wpT_ref[...].astype(jnp.bfloat16)
    bp = bp_ref[...]
    ctxr = ctx_ref[...]
    es = []
    for t in range(T):
        oc = jnp.concatenate([outs_f[t], outs_b[t]], axis=1)
        p = jnp.tanh(jnp.dot(oc, wpTb, preferred_element_type=jnp.float32)
                     + bp)
        s_t = jnp.sum(p * ctxr, axis=1, keepdims=True)
        es.append(jnp.exp(s_t))
    den = es[0]
    for t in range(1, T):
        den = den + es[t]
    inv = 1.0 / den
    att0 = es[0] * inv
    acc_f = outs_f[0].astype(jnp.float32) * att0
    acc_b = outs_b[0].astype(jnp.float32) * att0
    for t in range(1, T):
        att_t = es[t] * inv
        acc_f = acc_f + outs_f[t].astype(jnp.float32) * att_t
        acc_b = acc_b + outs_b[t].astype(jnp.float32) * att_t
    out_ref[...] = jnp.concatenate([acc_f, acc_b], axis=1)


@functools.partial(jax.jit, static_argnames=("interpret",))
def _char_to_word(padded_char_tensor, sequence_lens, emb, Wih_f, Whh_f,
                  bih_f, bhh_f, Wih_b, Whh_b, bih_b, bhh_b, Wp, bp, ctx,
                  interpret=False):
    NW, T = padded_char_tensor.shape
    V, EMB = emb.shape
    H = Whh_f.shape[1]
    C = Wp.shape[0]
    BW = 1024 if NW % 1024 == 0 else NW
    n_blocks = NW // BW

    chars2 = padded_char_tensor.astype(jnp.int32)
    lens2 = sequence_lens.astype(jnp.int32)[:, None]
    wihT = jnp.concatenate([Wih_f.T, Wih_b.T], axis=1)
    bih = jnp.concatenate([bih_f, bih_b])[None, :]
    out = pl.pallas_call(
        _block_kernel,
        out_shape=jax.ShapeDtypeStruct((NW, 2 * H), jnp.float32),
        grid=(n_blocks,),
        in_specs=[
            pl.BlockSpec((BW, T), lambda i: (i, 0)),
            pl.BlockSpec((BW, 1), lambda i: (i, 0)),
            pl.BlockSpec((V, EMB), lambda i: (0, 0)),
            pl.BlockSpec((EMB, 6 * H), lambda i: (0, 0)),
            pl.BlockSpec((H, 3 * H), lambda i: (0, 0)),
            pl.BlockSpec((H, 3 * H), lambda i: (0, 0)),
            pl.BlockSpec((1, 6 * H), lambda i: (0, 0)),
            pl.BlockSpec((1, 3 * H), lambda i: (0, 0)),
            pl.BlockSpec((1, 3 * H), lambda i: (0, 0)),
            pl.BlockSpec((2 * H, C), lambda i: (0, 0)),
            pl.BlockSpec((1, C), lambda i: (0, 0)),
            pl.BlockSpec((1, C), lambda i: (0, 0)),
        ],
        out_specs=pl.BlockSpec((BW, 2 * H), lambda i: (i, 0)),
        compiler_params=pltpu.CompilerParams(
            dimension_semantics=("parallel",),
            vmem_limit_bytes=56 * 1024 * 1024,
        ),
        name="char_to_word",
        interpret=interpret,
    )(
        chars2, lens2, emb, wihT, Whh_f.T, Whh_b.T, bih,
        bhh_f[None, :], bhh_b[None, :], Wp.T, bp[None, :], ctx.T,
    )
    return out


def kernel(padded_char_tensor, sequence_lens, emb, Wih_f, Whh_f, bih_f,
           bhh_f, Wih_b, Whh_b, bih_b, bhh_b, Wp, bp, ctx):
    return _char_to_word(padded_char_tensor, sequence_lens, emb, Wih_f,
                         Whh_f, bih_f, bhh_f, Wih_b, Whh_b, bih_b, bhh_b,
                         Wp, bp, ctx)
